# baseline (speedup 1.0000x reference)
import functools
import jax
import jax.numpy as jnp
from jax import lax
from jax.experimental import pallas as pl
from jax.experimental.pallas import tpu as pltpu
from jax.experimental.pallas import tpu_sc as plsc

_N = 10000
_E = 320000
_D = 128
_NS = 16
_TPW = _E // _NS
_K = 80
_NCH = _TPW // _K
_NP = 10240
_RPT = _NP // _NS
_ZR = 128
_B = 1000
_NB = 2 * _N // _B
_NBV = _N // _B


def _sc_mesh():
    return plsc.VectorSubcoreMesh(core_axis_name="c", subcore_axis_name="s")


_RP = 2
_RIX = 5
_RS = 5
_RID = 10
_ZRO = 64


@functools.partial(
    pl.kernel,
    out_type=jax.ShapeDtypeStruct((2, _NP, _D), jnp.float32),
    mesh=_sc_mesh(),
    scratch_types=[
        [pltpu.VMEM((_K,), jnp.int32) for _ in range(_RID)],
        pltpu.VMEM((_K, _D), jnp.float32),
        pltpu.VMEM((_ZRO, _D), jnp.float32),
        pltpu.VMEM_SHARED((_NP, _D), jnp.float32),
        [pltpu.SemaphoreType.DMA for _ in range(_RID)],
        [pltpu.SemaphoreType.DMA for _ in range(_RS)],
    ],
)
def _deg_sc(dst_hbm, out_hbm, idxb, ones_v, stage_v, acc_sh, sis, sss):
    c = lax.axis_index("c")
    s = lax.axis_index("s")

    def ofill(i, _):
        ones_v[i // 8, pl.ds((i % 8) * 16, 16)] = jnp.ones((16,), jnp.float32)
        return 0

    lax.fori_loop(0, _K * 8, ofill, 0)

    def zrow(i, _):
        stage_v[i // 8, pl.ds((i % 8) * 16, 16)] = jnp.zeros((16,), jnp.float32)
        return 0

    lax.fori_loop(0, _ZRO * 8, zrow, 0)
    base_rows = s * _RPT
    for j in range(_RPT // _ZRO):
        pltpu.sync_copy(stage_v, acc_sh.at[pl.ds(base_rows + j * _ZRO, _ZRO)])
    plsc.subcore_barrier()

    def idx_start(g, q):
        pltpu.async_copy(dst_hbm.at[c, s, g], idxb[q], sis[q])

    def idx_wait(q):
        pltpu.make_async_copy(dst_hbm.at[0, 0, 0], idxb[q], sis[q]).wait()

    def scat_start(q, b):
        pltpu.async_copy(ones_v, acc_sh.at[idxb[q]], sss[b], add=True)

    def scat_wait(b):
        pltpu.make_async_copy(ones_v, acc_sh.at[idxb[0]], sss[b]).wait()

    idx_start(0, 0)
    idx_start(1, 1)

    @pl.loop(0, _NCH // _RID)
    def _(i):
        for u in range(_RID):
            g = i * _RID + u
            b = u % _RS
            q = u % _RID

            @pl.when(g >= _RS)
            def _():
                scat_wait(b)

            idx_wait(q)

            @pl.when(g + 2 < _NCH)
            def _():
                idx_start(g + 2, (u + 2) % _RID)

            scat_start(q, b)

    for b in range(_RS):
        scat_wait(b)
    plsc.subcore_barrier()

    for j in range(_RPT // _ZRO):
        off = base_rows + j * _ZRO
        pltpu.sync_copy(acc_sh.at[pl.ds(off, _ZRO)], stage_v)
        pltpu.sync_copy(stage_v, out_hbm.at[c, pl.ds(off, _ZRO)])


@functools.partial(
    pl.kernel,
    out_type=jax.ShapeDtypeStruct((2, _NP, _D), jnp.float32),
    mesh=_sc_mesh(),
    scratch_types=[
        [pltpu.VMEM((2, _K), jnp.int32) for _ in range(_RIX)],
        [pltpu.VMEM((_K, _D), jnp.float32) for _ in range(_RP)],
        pltpu.VMEM((_ZRO, _D), jnp.float32),
        pltpu.VMEM_SHARED((_NP, _D), jnp.float32),
        [pltpu.SemaphoreType.DMA for _ in range(_RIX)],
        [pltpu.SemaphoreType.DMA for _ in range(_RP)],
        [pltpu.SemaphoreType.DMA for _ in range(_RP)],
    ],
)
def _prop_sc(g_hbm, ids_hbm, out_hbm, idxb, rows, stage_v, acc_sh,
             sis, sgs, sss):
    c = lax.axis_index("c")
    s = lax.axis_index("s")

    def zrow(i, _):
        stage_v[i // 8, pl.ds((i % 8) * 16, 16)] = jnp.zeros((16,), jnp.float32)
        return 0

    lax.fori_loop(0, _ZRO * 8, zrow, 0)
    base_rows = s * _RPT
    for j in range(_RPT // _ZRO):
        pltpu.sync_copy(stage_v, acc_sh.at[pl.ds(base_rows + j * _ZRO, _ZRO)])
    plsc.subcore_barrier()

    def idx_start(g, q):
        pltpu.async_copy(ids_hbm.at[c, s, g], idxb[q], sis[q])

    def idx_wait(q):
        pltpu.make_async_copy(ids_hbm.at[0, 0, 0], idxb[q], sis[q]).wait()

    def gather_start(q, b):
        pltpu.async_copy(g_hbm.at[idxb[q].at[0]], rows[b], sgs[b])

    def gather_wait(b):
        pltpu.make_async_copy(g_hbm.at[idxb[0].at[0]], rows[b], sgs[b]).wait()

    def scat_start(q, b):
        pltpu.async_copy(rows[b], acc_sh.at[idxb[q].at[1]], sss[b], add=True)

    def scat_wait(b):
        pltpu.make_async_copy(rows[b], acc_sh.at[idxb[0].at[1]], sss[b]).wait()

    idx_start(0, 0)
    idx_start(1, 1)
    _U = 10

    @pl.loop(0, _NCH // _U)
    def _(i):
        for u in range(_U):
            g = i * _U + u
            b = u % _RP
            q = u % _RIX

            @pl.when(g >= _RP)
            def _():
                scat_wait(b)

            idx_wait(q)

            @pl.when(g + 2 < _NCH)
            def _():
                idx_start(g + 2, (u + 2) % _RIX)

            gather_start(q, b)
            bp = (b - 1) % _RP
            qp = (q - 1) % _RIX

            @pl.when(g >= 1)
            def _():
                gather_wait(bp)
                scat_start(qp, bp)

    blast = (_NCH - 1) % _RP
    qlast = (_NCH - 1) % _RIX
    gather_wait(blast)
    scat_start(qlast, blast)
    for b in range(_RP):
        scat_wait(b)
    plsc.subcore_barrier()

    for j in range(_RPT // _ZRO):
        off = base_rows + j * _ZRO
        pltpu.sync_copy(acc_sh.at[pl.ds(off, _ZRO)], stage_v)
        pltpu.sync_copy(stage_v, out_hbm.at[c, pl.ds(off, _ZRO)])


def _dinv_of(deg_blk):
    return lax.rsqrt(deg_blk[:, 0:1] + 1.0)


def _mm1_body(x_ref, w_ref, deg_ref, g_ref):
    dinv = _dinv_of(deg_ref[...])
    h = jnp.dot(x_ref[...], w_ref[...], preferred_element_type=jnp.float32)
    g_ref[...] = h * dinv


def _mm2_body(acc_ref, g_ref, deg_ref, w_ref, b_ref, g2_ref):
    dinv = _dinv_of(deg_ref[...])
    l1 = dinv * (acc_ref[...] + g_ref[...]) + b_ref[...]
    h2 = jnp.dot(l1, w_ref[...], preferred_element_type=jnp.float32)
    g2_ref[...] = h2 * dinv


def _head_body(acc_ref, g_ref, deg_ref, b_ref, wp_ref, bp_ref,
               v_ref, p_ref, sum_ref, sq_ref):
    i = pl.program_id(0)
    dinv = _dinv_of(deg_ref[...])
    v = dinv * (acc_ref[...] + g_ref[...]) + b_ref[...]
    v_ref[...] = v
    p = jnp.dot(v, wp_ref[...], preferred_element_type=jnp.float32) + bp_ref[...]
    p_ref[...] = p
    ps = jnp.sum(p, axis=0, keepdims=True)
    sq = jnp.sum(p * p, axis=0, keepdims=True)

    @pl.when(i == 0)
    def _():
        sum_ref[...] = jnp.zeros((2, _D), jnp.float32)
        sq_ref[...] = jnp.zeros((2, _D), jnp.float32)

    @pl.when(i < _NBV)
    def _():
        sum_ref[0:1, :] += ps
        sq_ref[0:1, :] += sq

    @pl.when(i >= _NBV)
    def _():
        sum_ref[1:2, :] += ps
        sq_ref[1:2, :] += sq


def _bn_relu_norm(p, mean, var, gamma, beta):
    a = (p - mean) / jnp.sqrt(var + 1e-5) * gamma + beta
    a = jnp.maximum(a, 0.0)
    n = jnp.sqrt(jnp.sum(a * a, axis=-1, keepdims=True))
    return a / jnp.maximum(n, 1e-12)


def _rownorm(x):
    n = jnp.sqrt(jnp.sum(x * x, axis=-1, keepdims=True))
    return x / jnp.maximum(n, 1e-12)


def _loss_body(p1_ref, p2_ref, v1_ref, v2_ref, sum_ref, sq_ref,
               gamma_ref, beta_ref, loss_ref):
    i = pl.program_id(0)
    nf = jnp.float32(_N)
    mean1 = sum_ref[0:1, :] / nf
    mean2 = sum_ref[1:2, :] / nf
    var1 = sq_ref[0:1, :] / nf - mean1 * mean1
    var2 = sq_ref[1:2, :] / nf - mean2 * mean2
    gamma = gamma_ref[...]
    beta = beta_ref[...]
    a1 = _bn_relu_norm(p1_ref[...], mean1, var1, gamma, beta)
    a2 = _bn_relu_norm(p2_ref[...], mean2, var2, gamma, beta)
    t1 = _rownorm(v1_ref[...])
    t2 = _rownorm(v2_ref[...])
    l1 = 2.0 - 2.0 * jnp.sum(a1 * t2, axis=-1)
    l2 = 2.0 - 2.0 * jnp.sum(a2 * t1, axis=-1)
    part = jnp.sum(l1 + l2).reshape(1, 1)

    @pl.when(i == 0)
    def _():
        loss_ref[...] = part

    @pl.when(i != 0)
    def _():
        loss_ref[...] += part

    @pl.when(i == _NBV - 1)
    def _():
        loss_ref[...] = loss_ref[...] / nf


def kernel(x1, x2, edge_index_v1, edge_index_v2, W1, b1, W2, b2, Wp, bp,
           gamma, beta):
    f32 = jnp.float32
    x = jnp.concatenate([x1, x2], axis=0)
    src = jnp.concatenate(
        [edge_index_v1[0], edge_index_v2[0] + _N]).reshape(2, _NS, _NCH, _K)
    dst = jnp.concatenate(
        [edge_index_v1[1], edge_index_v2[1]]).reshape(2, _NS, _NCH, _K)
    ids = jnp.stack([src, dst], axis=3)

    degf = _deg_sc(dst)
    deg = degf[:, :_N, :1].reshape(2 * _N, 1)

    b1r = b1.reshape(1, _D)
    b2r = b2.reshape(1, _D)
    bpr = bp.reshape(1, _D)
    gammar = gamma.reshape(1, _D)
    betar = beta.reshape(1, _D)

    blk = lambda i: (i, 0)
    cst = lambda i: (0, 0)
    row_spec = pl.BlockSpec((_B, _D), blk)
    deg_spec = pl.BlockSpec((_B, 1), blk)
    w_spec = pl.BlockSpec((_D, _D), cst)
    b_spec = pl.BlockSpec((1, _D), cst)

    g1 = pl.pallas_call(
        _mm1_body,
        grid=(_NB,),
        in_specs=[row_spec, w_spec, deg_spec],
        out_specs=row_spec,
        out_shape=jax.ShapeDtypeStruct((2 * _N, _D), f32),
    )(x, W1, deg)

    acc1 = _prop_sc(g1, ids)[:, :_N, :].reshape(2 * _N, _D)

    g2 = pl.pallas_call(
        _mm2_body,
        grid=(_NB,),
        in_specs=[row_spec, row_spec, deg_spec, w_spec, b_spec],
        out_specs=row_spec,
        out_shape=jax.ShapeDtypeStruct((2 * _N, _D), f32),
    )(acc1, g1, deg, W2, b1r)

    acc2 = _prop_sc(g2, ids)[:, :_N, :].reshape(2 * _N, _D)

    stat_spec = pl.BlockSpec((2, _D), lambda i: (0, 0))
    v, p, psum, psq = pl.pallas_call(
        _head_body,
        grid=(_NB,),
        in_specs=[row_spec, row_spec, deg_spec, b_spec, w_spec, b_spec],
        out_specs=[row_spec, row_spec, stat_spec, stat_spec],
        out_shape=[
            jax.ShapeDtypeStruct((2 * _N, _D), f32),
            jax.ShapeDtypeStruct((2 * _N, _D), f32),
            jax.ShapeDtypeStruct((2, _D), f32),
            jax.ShapeDtypeStruct((2, _D), f32),
        ],
    )(acc2, g2, deg, b2r, Wp, bpr)

    v1_spec = pl.BlockSpec((_B, _D), lambda i: (i, 0))
    v2_spec = pl.BlockSpec((_B, _D), lambda i: (i + _NBV, 0))
    stat_full = pl.BlockSpec((2, _D), cst)
    loss = pl.pallas_call(
        _loss_body,
        grid=(_NBV,),
        in_specs=[v1_spec, v2_spec, v1_spec, v2_spec, stat_full, stat_full,
                  b_spec, b_spec],
        out_specs=pl.BlockSpec((1, 1), cst),
        out_shape=jax.ShapeDtypeStruct((1, 1), f32),
    )(p, p, v, v, psum, psq, gammar, betar)

    return (v[:_N], v[_N:], loss[0, 0])


# speedup vs baseline: 1.0053x; 1.0053x over previous
import functools
import jax
import jax.numpy as jnp
from jax import lax
from jax.experimental import pallas as pl
from jax.experimental.pallas import tpu as pltpu
from jax.experimental.pallas import tpu_sc as plsc

_N = 10000
_E = 320000
_D = 128
_NS = 16
_TPW = _E // _NS
_K = 80
_NCH = _TPW // _K
_NP = 10240
_RPT = _NP // _NS
_ZR = 128
_B = 1000
_NB = 2 * _N // _B
_NBV = _N // _B


def _sc_mesh():
    return plsc.VectorSubcoreMesh(core_axis_name="c", subcore_axis_name="s")


_RP = 2
_RIX = 5
_RS = 5
_RID = 10
_ZRO = 64


@functools.partial(
    pl.kernel,
    out_type=jax.ShapeDtypeStruct((2, _NP, _D), jnp.float32),
    mesh=_sc_mesh(),
    scratch_types=[
        [pltpu.VMEM((_K,), jnp.int32) for _ in range(_RID)],
        pltpu.VMEM((_K, _D), jnp.float32),
        pltpu.VMEM((_ZRO, _D), jnp.float32),
        pltpu.VMEM_SHARED((_NP, _D), jnp.float32),
        [pltpu.SemaphoreType.DMA for _ in range(_RID)],
        [pltpu.SemaphoreType.DMA for _ in range(_RS)],
    ],
)
def _deg_sc(dst_hbm, out_hbm, idxb, ones_v, stage_v, acc_sh, sis, sss):
    c = lax.axis_index("c")
    s = lax.axis_index("s")

    def ofill(i, _):
        ones_v[i // 8, pl.ds((i % 8) * 16, 16)] = jnp.ones((16,), jnp.float32)
        return 0

    lax.fori_loop(0, _K * 8, ofill, 0)

    def zrow(i, _):
        stage_v[i // 8, pl.ds((i % 8) * 16, 16)] = jnp.zeros((16,), jnp.float32)
        return 0

    lax.fori_loop(0, _ZRO * 8, zrow, 0)
    base_rows = s * _RPT
    for j in range(_RPT // _ZRO):
        pltpu.sync_copy(stage_v, acc_sh.at[pl.ds(base_rows + j * _ZRO, _ZRO)])
    plsc.subcore_barrier()

    def idx_start(g, q):
        pltpu.async_copy(dst_hbm.at[c, s, g], idxb[q], sis[q])

    def idx_wait(q):
        pltpu.make_async_copy(dst_hbm.at[0, 0, 0], idxb[q], sis[q]).wait()

    def scat_start(q, b):
        pltpu.async_copy(ones_v, acc_sh.at[idxb[q]], sss[b], add=True)

    def scat_wait(b):
        pltpu.make_async_copy(ones_v, acc_sh.at[idxb[0]], sss[b]).wait()

    idx_start(0, 0)
    idx_start(1, 1)

    @pl.loop(0, _NCH // _RID)
    def _(i):
        for u in range(_RID):
            g = i * _RID + u
            b = u % _RS
            q = u % _RID

            @pl.when(g >= _RS)
            def _():
                scat_wait(b)

            idx_wait(q)

            @pl.when(g + 2 < _NCH)
            def _():
                idx_start(g + 2, (u + 2) % _RID)

            scat_start(q, b)

    for b in range(_RS):
        scat_wait(b)
    plsc.subcore_barrier()

    for j in range(_RPT // _ZRO):
        off = base_rows + j * _ZRO
        pltpu.sync_copy(acc_sh.at[pl.ds(off, _ZRO)], stage_v)
        pltpu.sync_copy(stage_v, out_hbm.at[c, pl.ds(off, _ZRO)])


@functools.partial(
    pl.kernel,
    out_type=jax.ShapeDtypeStruct((2, _NP, _D), jnp.float32),
    mesh=_sc_mesh(),
    scratch_types=[
        [pltpu.VMEM((2, _K), jnp.int32) for _ in range(_RIX)],
        [pltpu.VMEM((_K, _D), jnp.float32) for _ in range(_RP)],
        pltpu.VMEM((_ZRO, _D), jnp.float32),
        pltpu.VMEM_SHARED((_NP, _D), jnp.float32),
        [pltpu.SemaphoreType.DMA for _ in range(_RIX)],
        [pltpu.SemaphoreType.DMA for _ in range(_RP)],
        [pltpu.SemaphoreType.DMA for _ in range(_RP)],
    ],
)
def _prop_sc(g_hbm, ids_hbm, out_hbm, idxb, rows, stage_v, acc_sh,
             sis, sgs, sss):
    c = lax.axis_index("c")
    s = lax.axis_index("s")

    def zrow(i, _):
        stage_v[i // 8, pl.ds((i % 8) * 16, 16)] = jnp.zeros((16,), jnp.float32)
        return 0

    lax.fori_loop(0, _ZRO * 8, zrow, 0)
    base_rows = s * _RPT
    for j in range(_RPT // _ZRO):
        pltpu.sync_copy(stage_v, acc_sh.at[pl.ds(base_rows + j * _ZRO, _ZRO)])
    plsc.subcore_barrier()

    def idx_start(g, q):
        pltpu.async_copy(ids_hbm.at[c, s, g], idxb[q], sis[q])

    def idx_wait(q):
        pltpu.make_async_copy(ids_hbm.at[0, 0, 0], idxb[q], sis[q]).wait()

    def gather_start(q, b):
        pltpu.async_copy(g_hbm.at[idxb[q].at[0]], rows[b], sgs[b])

    def gather_wait(b):
        pltpu.make_async_copy(g_hbm.at[idxb[0].at[0]], rows[b], sgs[b]).wait()

    def scat_start(q, b):
        pltpu.async_copy(rows[b], acc_sh.at[idxb[q].at[1]], sss[b], add=True)

    def scat_wait(b):
        pltpu.make_async_copy(rows[b], acc_sh.at[idxb[0].at[1]], sss[b]).wait()

    idx_start(0, 0)
    idx_start(1, 1)
    _U = 10

    @pl.loop(0, _NCH // _U)
    def _(i):
        for u in range(_U):
            g = i * _U + u
            b = u % _RP
            q = u % _RIX

            @pl.when(g >= _RP)
            def _():
                scat_wait(b)

            idx_wait(q)

            @pl.when(g + 2 < _NCH)
            def _():
                idx_start(g + 2, (u + 2) % _RIX)

            gather_start(q, b)
            bp = (b - 1) % _RP
            qp = (q - 1) % _RIX

            @pl.when(g >= 1)
            def _():
                gather_wait(bp)
                scat_start(qp, bp)

    blast = (_NCH - 1) % _RP
    qlast = (_NCH - 1) % _RIX
    gather_wait(blast)
    scat_start(qlast, blast)
    for b in range(_RP):
        scat_wait(b)
    plsc.subcore_barrier()

    for j in range(_RPT // _ZRO):
        off = base_rows + j * _ZRO
        pltpu.sync_copy(acc_sh.at[pl.ds(off, _ZRO)], stage_v)
        pltpu.sync_copy(stage_v, out_hbm.at[c, pl.ds(off, _ZRO)])


def _dinv_of(deg_blk):
    return lax.rsqrt(deg_blk[:, 0:1] + 1.0)


def _mm1_body(x_ref, w_ref, deg_ref, g_ref):
    dinv = _dinv_of(deg_ref[...])
    h = jnp.dot(x_ref[...], w_ref[...], preferred_element_type=jnp.float32)
    g_ref[...] = h * dinv


def _mm2_body(acc_ref, g_ref, deg_ref, w_ref, b_ref, g2_ref):
    dinv = _dinv_of(deg_ref[...])
    l1 = dinv * (acc_ref[...] + g_ref[...]) + b_ref[...]
    h2 = jnp.dot(l1, w_ref[...], preferred_element_type=jnp.float32)
    g2_ref[...] = h2 * dinv


def _bn_relu_norm(p, mean, var, gamma, beta):
    a = (p - mean) / jnp.sqrt(var + 1e-5) * gamma + beta
    a = jnp.maximum(a, 0.0)
    n = jnp.sqrt(jnp.sum(a * a, axis=-1, keepdims=True))
    return a / jnp.maximum(n, 1e-12)


def _rownorm(x):
    n = jnp.sqrt(jnp.sum(x * x, axis=-1, keepdims=True))
    return x / jnp.maximum(n, 1e-12)


def _headloss_body(acc_ref, g_ref, deg_ref, b_ref, wp_ref, bp_ref,
                   gamma_ref, beta_ref, v_ref, loss_ref,
                   pbuf, vbuf, sum_s, sq_s):
    i = pl.program_id(0)
    nf = jnp.float32(_N)

    @pl.when(i < _NB)
    def _():
        dinv = _dinv_of(deg_ref[...])
        v = dinv * (acc_ref[...] + g_ref[...]) + b_ref[...]
        v_ref[...] = v
        vbuf[pl.ds(i * _B, _B), :] = v
        p = (jnp.dot(v, wp_ref[...], preferred_element_type=jnp.float32)
             + bp_ref[...])
        pbuf[pl.ds(i * _B, _B), :] = p
        ps = jnp.sum(p, axis=0, keepdims=True)
        sq = jnp.sum(p * p, axis=0, keepdims=True)

        @pl.when(i == 0)
        def _():
            sum_s[...] = jnp.zeros((2, _D), jnp.float32)
            sq_s[...] = jnp.zeros((2, _D), jnp.float32)

        @pl.when(i < _NBV)
        def _():
            sum_s[0:1, :] += ps
            sq_s[0:1, :] += sq

        @pl.when(jnp.logical_and(i >= _NBV, i < _NB))
        def _():
            sum_s[1:2, :] += ps
            sq_s[1:2, :] += sq

    @pl.when(i >= _NB)
    def _():
        j = i - _NB
        mean1 = sum_s[0:1, :] / nf
        mean2 = sum_s[1:2, :] / nf
        var1 = sq_s[0:1, :] / nf - mean1 * mean1
        var2 = sq_s[1:2, :] / nf - mean2 * mean2
        p1 = pbuf[pl.ds(j * _B, _B), :]
        p2 = pbuf[pl.ds((j + _NBV) * _B, _B), :]
        t1 = _rownorm(vbuf[pl.ds(j * _B, _B), :])
        t2 = _rownorm(vbuf[pl.ds((j + _NBV) * _B, _B), :])
        gamma = gamma_ref[...]
        beta = beta_ref[...]
        a1 = _bn_relu_norm(p1, mean1, var1, gamma, beta)
        a2 = _bn_relu_norm(p2, mean2, var2, gamma, beta)
        l1 = 2.0 - 2.0 * jnp.sum(a1 * t2, axis=-1)
        l2 = 2.0 - 2.0 * jnp.sum(a2 * t1, axis=-1)
        part = jnp.sum(l1 + l2).reshape(1, 1)

        @pl.when(j == 0)
        def _():
            loss_ref[...] = part

        @pl.when(j > 0)
        def _():
            loss_ref[...] += part

        @pl.when(j == _NBV - 1)
        def _():
            loss_ref[...] = loss_ref[...] / nf


def kernel(x1, x2, edge_index_v1, edge_index_v2, W1, b1, W2, b2, Wp, bp,
           gamma, beta):
    f32 = jnp.float32
    x = jnp.concatenate([x1, x2], axis=0)
    src = jnp.concatenate(
        [edge_index_v1[0], edge_index_v2[0] + _N]).reshape(2, _NS, _NCH, _K)
    dst = jnp.concatenate(
        [edge_index_v1[1], edge_index_v2[1]]).reshape(2, _NS, _NCH, _K)
    ids = jnp.stack([src, dst], axis=3)

    degf = _deg_sc(dst)
    deg = degf[:, :_N, :1].reshape(2 * _N, 1)

    b1r = b1.reshape(1, _D)
    b2r = b2.reshape(1, _D)
    bpr = bp.reshape(1, _D)
    gammar = gamma.reshape(1, _D)
    betar = beta.reshape(1, _D)

    blk = lambda i: (i, 0)
    cst = lambda i: (0, 0)
    row_spec = pl.BlockSpec((_B, _D), blk)
    deg_spec = pl.BlockSpec((_B, 1), blk)
    w_spec = pl.BlockSpec((_D, _D), cst)
    b_spec = pl.BlockSpec((1, _D), cst)

    g1 = pl.pallas_call(
        _mm1_body,
        grid=(_NB,),
        in_specs=[row_spec, w_spec, deg_spec],
        out_specs=row_spec,
        out_shape=jax.ShapeDtypeStruct((2 * _N, _D), f32),
    )(x, W1, deg)

    acc1 = _prop_sc(g1, ids)[:, :_N, :].reshape(2 * _N, _D)

    g2 = pl.pallas_call(
        _mm2_body,
        grid=(_NB,),
        in_specs=[row_spec, row_spec, deg_spec, w_spec, b_spec],
        out_specs=row_spec,
        out_shape=jax.ShapeDtypeStruct((2 * _N, _D), f32),
    )(acc1, g1, deg, W2, b1r)

    acc2 = _prop_sc(g2, ids)[:, :_N, :].reshape(2 * _N, _D)

    hl_row = pl.BlockSpec((_B, _D), lambda i: (jnp.minimum(i, _NB - 1), 0))
    hl_deg = pl.BlockSpec((_B, 1), lambda i: (jnp.minimum(i, _NB - 1), 0))
    v, loss = pl.pallas_call(
        _headloss_body,
        grid=(_NB + _NBV,),
        in_specs=[hl_row, hl_row, hl_deg, b_spec, w_spec, b_spec, b_spec,
                  b_spec],
        out_specs=[hl_row, pl.BlockSpec((1, 1), cst)],
        out_shape=[
            jax.ShapeDtypeStruct((2 * _N, _D), f32),
            jax.ShapeDtypeStruct((1, 1), f32),
        ],
        scratch_shapes=[
            pltpu.VMEM((2 * _N, _D), f32),
            pltpu.VMEM((2 * _N, _D), f32),
            pltpu.VMEM((2, _D), f32),
            pltpu.VMEM((2, _D), f32),
        ],
    )(acc2, g2, deg, b2r, Wp, bpr, gammar, betar)

    return (v[:_N], v[_N:], loss[0, 0])
